# Initial kernel scaffold; baseline (speedup 1.0000x reference)
import jax
import jax.numpy as jnp
from jax.experimental import pallas as pl


def kernel(x, xlens, W1, b1, Wr, br, Wz, bz):
    raise NotImplementedError("write your pallas kernel here")


# speedup vs baseline: 3.0386x; 3.0386x over previous
import functools
import math

import jax
import jax.numpy as jnp
from jax import lax
from jax.experimental import pallas as pl
from jax.experimental.pallas import tpu as pltpu
from jax.experimental.pallas import tpu_sc as plsc

B = 16
L = 4096
N = 32768
D = 64
H = 32
LANES = 16

SCALE = 400.0 / math.log(10.0)

ROWS = 2048
WIN = 4104
NP = 36864


def _mlp_body(x_ref, w1_ref, b1_ref, wrz_ref, r_ref, z_ref):
    h = jnp.dot(x_ref[...], w1_ref[...], preferred_element_type=jnp.float32)
    h = jnp.maximum(h + b1_ref[...][None, :], 0.0)
    rz = jnp.dot(h, wrz_ref[...], preferred_element_type=jnp.float32)
    r_ref[...] = rz[:, 0]
    z_ref[...] = rz[:, 1]


def _tc_mlp(x, W1, b1, Wrz):
    nblk = N // ROWS
    return pl.pallas_call(
        _mlp_body,
        grid=(nblk,),
        in_specs=[
            pl.BlockSpec((ROWS, D), lambda i: (i, 0)),
            pl.BlockSpec((D, H), lambda i: (0, 0)),
            pl.BlockSpec((H,), lambda i: (0,)),
            pl.BlockSpec((H, 2), lambda i: (0, 0)),
        ],
        out_specs=[
            pl.BlockSpec((ROWS,), lambda i: (i,)),
            pl.BlockSpec((ROWS,), lambda i: (i,)),
        ],
        out_shape=[
            jax.ShapeDtypeStruct((NP,), jnp.float32),
            jax.ShapeDtypeStruct((NP,), jnp.float32),
        ],
    )(x, W1, b1, Wrz)


def _lane_iota():
    return lax.iota(jnp.int32, LANES)


_GAT_DNUMS = lax.GatherDimensionNumbers(
    offset_dims=(), collapsed_slice_dims=(0,), start_index_map=(0,))


def _gat(x, idx):
    return lax.gather(x, idx[:, None], _GAT_DNUMS, slice_sizes=(1,),
                      mode=lax.GatherScatterMode.PROMISE_IN_BOUNDS)


def _bfly_sum(x):
    lane = _lane_iota()
    for k in (8, 4, 2, 1):
        x = x + _gat(x, lane ^ k)
    return x


def _bfly_max(x):
    lane = _lane_iota()
    for k in (8, 4, 2, 1):
        x = jnp.maximum(x, _gat(x, lane ^ k))
    return x


def _prefix_sum(x):
    lane = _lane_iota()
    for k in (1, 2, 4, 8):
        sh = _gat(x, jnp.maximum(lane - k, 0))
        x = x + jnp.where(lane >= k, sh, jnp.zeros_like(x))
    return x


def _sc_pool(r, z, xlens, brv):
    mesh = plsc.VectorSubcoreMesh(core_axis_name="c", subcore_axis_name="s")

    @functools.partial(
        pl.kernel,
        out_type=jax.ShapeDtypeStruct((B, LANES), jnp.float32),
        mesh=mesh,
        scratch_types=[
            pltpu.VMEM((LANES,), jnp.int32),
            pltpu.VMEM((LANES,), jnp.float32),
            pltpu.VMEM((WIN,), jnp.float32),
            pltpu.VMEM((WIN,), jnp.float32),
            pltpu.VMEM((LANES,), jnp.float32),
        ],
    )
    def k(r_hbm, z_hbm, xl_hbm, br_hbm, out_hbm,
          xl_v, br_v, z_v, r_v, o_v):
        c = lax.axis_index("c")
        s = lax.axis_index("s")

        @pl.when(c == 0)
        def _():
            pltpu.sync_copy(xl_hbm, xl_v)
            pltpu.sync_copy(br_hbm, br_v)
            xl = xl_v[...].astype(jnp.float32)
            incl_raw = _prefix_sum(xl)
            cli = jnp.minimum(incl_raw, float(N))
            cle = jnp.minimum(incl_raw - xl, float(N))
            sel = _lane_iota() == s
            zero = jnp.zeros((LANES,), jnp.float32)
            hi = _bfly_sum(jnp.where(sel, cli, zero))[0].astype(jnp.int32)
            lo = _bfly_sum(jnp.where(sel, cle, zero))[0].astype(jnp.int32)

            @pl.when(hi > lo)
            def _nonempty():
                start0 = pl.multiple_of(lo & (-8), 8)
                pltpu.sync_copy(z_hbm.at[pl.ds(start0, WIN)], z_v)
                pltpu.sync_copy(r_hbm.at[pl.ds(start0, WIN)], r_v)
                nvec = (hi - start0 + (LANES - 1)) // LANES
                lane = lax.iota(jnp.int32, LANES)

                def max_body(v, mcur):
                    idx = (start0 + v * LANES) + lane
                    zv = z_v[pl.ds(v * LANES, LANES)]
                    msk = (idx >= lo) & (idx < hi)
                    return jnp.maximum(mcur, jnp.where(msk, zv, -jnp.inf))

                mvec = lax.fori_loop(
                    0, nvec, max_body,
                    jnp.full((LANES,), -jnp.inf, jnp.float32))
                msegv = _bfly_max(mvec)

                def sum_body(v, carry):
                    sv, nv = carry
                    idx = (start0 + v * LANES) + lane
                    zv = z_v[pl.ds(v * LANES, LANES)]
                    rv = r_v[pl.ds(v * LANES, LANES)]
                    msk = (idx >= lo) & (idx < hi)
                    ez = jnp.where(msk, jnp.exp(zv - msegv), 0.0)
                    return (sv + ez, nv + ez * rv)

                sv, nv = lax.fori_loop(
                    0, nvec, sum_body,
                    (jnp.zeros((LANES,), jnp.float32),
                     jnp.zeros((LANES,), jnp.float32)))
                ssum = _bfly_sum(sv)
                nsum = _bfly_sum(nv)
                o_v[...] = SCALE * (nsum / ssum + br_v[...])
                pltpu.sync_copy(o_v, out_hbm.at[s])

            @pl.when(hi <= lo)
            def _empty():
                o_v[...] = jnp.zeros((LANES,), jnp.float32)
                pltpu.sync_copy(o_v, out_hbm.at[s])

    return k(r, z, xlens, brv)


def kernel(x, xlens, W1, b1, Wr, br, Wz, bz):
    del bz
    Wrz = jnp.concatenate([Wr, Wz], axis=1)
    r, z = _tc_mlp(x, W1, b1, Wrz)
    brv = jnp.broadcast_to(br.astype(jnp.float32), (LANES,))
    pooled = _sc_pool(r, z, xlens.astype(jnp.int32), brv)
    return pooled[:, 0]


# speedup vs baseline: 3.3015x; 1.0865x over previous
import functools
import math

import jax
import jax.numpy as jnp
from jax import lax
from jax.experimental import pallas as pl
from jax.experimental.pallas import tpu as pltpu
from jax.experimental.pallas import tpu_sc as plsc

B = 16
L = 4096
N = 32768
D = 64
H = 32
LANES = 16

SCALE = 400.0 / math.log(10.0)

ROWS = 2048
WIN = 4104
NP = 36864


def _mlp_body(x_ref, w1_ref, b1_ref, wr_ref, wz_ref, br_ref, r_ref, z_ref):
    h = jnp.dot(x_ref[...], w1_ref[...], preferred_element_type=jnp.float32)
    h = jnp.maximum(h + b1_ref[...][None, :], 0.0)
    wrz = jnp.concatenate([wr_ref[...], wz_ref[...]], axis=1)
    rzt = lax.dot_general(wrz, h, (((0,), (1,)), ((), ())),
                          preferred_element_type=jnp.float32)
    r_ref[...] = rzt[0] + br_ref[0]
    z_ref[...] = rzt[1]


def _tc_mlp(x, W1, b1, Wr, Wz, br):
    nblk = N // ROWS
    return pl.pallas_call(
        _mlp_body,
        grid=(nblk,),
        in_specs=[
            pl.BlockSpec((ROWS, D), lambda i: (i, 0)),
            pl.BlockSpec((D, H), lambda i: (0, 0)),
            pl.BlockSpec((H,), lambda i: (0,)),
            pl.BlockSpec((H, 1), lambda i: (0, 0)),
            pl.BlockSpec((H, 1), lambda i: (0, 0)),
            pl.BlockSpec((1,), lambda i: (0,)),
        ],
        out_specs=[
            pl.BlockSpec((ROWS,), lambda i: (i,)),
            pl.BlockSpec((ROWS,), lambda i: (i,)),
        ],
        out_shape=[
            jax.ShapeDtypeStruct((NP,), jnp.float32),
            jax.ShapeDtypeStruct((NP,), jnp.float32),
        ],
    )(x, W1, b1, Wr, Wz, br)


def _lane_iota():
    return lax.iota(jnp.int32, LANES)


_GAT_DNUMS = lax.GatherDimensionNumbers(
    offset_dims=(), collapsed_slice_dims=(0,), start_index_map=(0,))


def _gat(x, idx):
    return lax.gather(x, idx[:, None], _GAT_DNUMS, slice_sizes=(1,),
                      mode=lax.GatherScatterMode.PROMISE_IN_BOUNDS)


def _bfly_sum(x):
    lane = _lane_iota()
    for k in (8, 4, 2, 1):
        x = x + _gat(x, lane ^ k)
    return x


def _bfly_max(x):
    lane = _lane_iota()
    for k in (8, 4, 2, 1):
        x = jnp.maximum(x, _gat(x, lane ^ k))
    return x


def _prefix_sum(x):
    lane = _lane_iota()
    for k in (1, 2, 4, 8):
        sh = _gat(x, jnp.maximum(lane - k, 0))
        x = x + jnp.where(lane >= k, sh, jnp.zeros_like(x))
    return x


def _sc_pool(r, z, xlens):
    mesh = plsc.VectorSubcoreMesh(core_axis_name="c", subcore_axis_name="s")

    @functools.partial(
        pl.kernel,
        out_type=jax.ShapeDtypeStruct((B, LANES), jnp.float32),
        mesh=mesh,
        scratch_types=[
            pltpu.VMEM((LANES,), jnp.int32),
            pltpu.VMEM((WIN,), jnp.float32),
            pltpu.VMEM((WIN,), jnp.float32),
            pltpu.VMEM((LANES,), jnp.float32),
        ],
    )
    def k(r_hbm, z_hbm, xl_hbm, out_hbm, xl_v, r_v, z_v, o_v):
        c = lax.axis_index("c")
        s = lax.axis_index("s")

        @pl.when(c == 0)
        def _():
            pltpu.sync_copy(xl_hbm, xl_v)
            xl = xl_v[...].astype(jnp.float32)
            incl_raw = _prefix_sum(xl)
            cli = jnp.minimum(incl_raw, float(N))
            cle = jnp.minimum(incl_raw - xl, float(N))
            sel = _lane_iota() == s
            zero = jnp.zeros((LANES,), jnp.float32)
            hi = _bfly_sum(jnp.where(sel, cli, zero))[0].astype(jnp.int32)
            lo = _bfly_sum(jnp.where(sel, cle, zero))[0].astype(jnp.int32)

            @pl.when(hi > lo)
            def _nonempty():
                start0 = pl.multiple_of(lo & (-8), 8)
                pltpu.sync_copy(r_hbm.at[pl.ds(start0, WIN)], r_v)
                pltpu.sync_copy(z_hbm.at[pl.ds(start0, WIN)], z_v)
                nvec = (hi - start0 + (LANES - 1)) // LANES
                lane = _lane_iota()

                def max_body(v, mcur):
                    idx = (start0 + v * LANES) + lane
                    zv = z_v[pl.ds(v * LANES, LANES)]
                    msk = (idx >= lo) & (idx < hi)
                    return jnp.maximum(mcur, jnp.where(msk, zv, -jnp.inf))

                mvec = lax.fori_loop(
                    0, nvec, max_body,
                    jnp.full((LANES,), -jnp.inf, jnp.float32))
                msegv = _bfly_max(mvec)

                def sum_body(v, carry):
                    sv, nv = carry
                    idx = (start0 + v * LANES) + lane
                    zv = z_v[pl.ds(v * LANES, LANES)]
                    rv = r_v[pl.ds(v * LANES, LANES)]
                    msk = (idx >= lo) & (idx < hi)
                    ez = jnp.where(msk, jnp.exp(zv - msegv), 0.0)
                    return (sv + ez, nv + ez * rv)

                sv, nv = lax.fori_loop(
                    0, nvec, sum_body,
                    (jnp.zeros((LANES,), jnp.float32),
                     jnp.zeros((LANES,), jnp.float32)))
                ssum = _bfly_sum(sv)
                nsum = _bfly_sum(nv)
                o_v[...] = SCALE * (nsum / ssum)
                pltpu.sync_copy(o_v, out_hbm.at[s])

            @pl.when(hi <= lo)
            def _empty():
                o_v[...] = jnp.zeros((LANES,), jnp.float32)
                pltpu.sync_copy(o_v, out_hbm.at[s])

    return k(r, z, xlens)


def kernel(x, xlens, W1, b1, Wr, br, Wz, bz):
    del bz
    r, z = _tc_mlp(x, W1, b1, Wr, Wz, br)
    pooled = _sc_pool(r, z, xlens)
    return pooled[:, 0]
